# baseline (speedup 1.0000x reference)
import functools

import jax
import jax.numpy as jnp
from jax import lax
from jax.experimental import pallas as pl
from jax.experimental.pallas import tpu as pltpu
from jax.experimental.pallas import tpu_sc as plsc

NC = 2
NS = 16
NW = NC * NS


@functools.lru_cache(maxsize=None)
def _build(n_idx: int, V: int, D: int, chunk: int, nbuf: int):
    assert n_idx % NW == 0
    b_per_w = n_idx // NW
    assert b_per_w % chunk == 0 and chunk % 8 == 0
    nchunks = b_per_w // chunk

    mesh = plsc.VectorSubcoreMesh(
        core_axis_name="c", subcore_axis_name="s",
        num_cores=NC, num_subcores=NS,
    )

    @functools.partial(
        pl.kernel,
        out_type=jax.ShapeDtypeStruct((n_idx, D), jnp.float32),
        mesh=mesh,
        scratch_types=[
            pltpu.VMEM((b_per_w,), jnp.int32),
            pltpu.VMEM((nbuf, chunk, D), jnp.float32),
            [pltpu.SemaphoreType.DMA] * nbuf,
            [pltpu.SemaphoreType.DMA] * nbuf,
        ],
        compiler_params=pltpu.CompilerParams(use_tc_tiling_on_sc=False),
    )
    def gather_kernel(idx_hbm, table_hbm, out_hbm, idx_v, rows_v, gsems, wsems):
        wid = lax.axis_index("s") * NC + lax.axis_index("c")
        base = wid * b_per_w
        pltpu.sync_copy(idx_hbm.at[pl.ds(base, b_per_w)], idx_v)

        def gather(c, b):
            return pltpu.async_copy(
                table_hbm.at[idx_v.at[pl.ds(c * chunk, chunk)]],
                rows_v.at[b], gsems[b])

        gds = [gather(b, b) for b in range(nbuf)]
        for c in range(nchunks):
            b = c % nbuf
            gds[b].wait()
            wd = pltpu.async_copy(
                rows_v.at[b], out_hbm.at[pl.ds(base + c * chunk, chunk)],
                wsems[b])
            wd.wait()
            if c + nbuf < nchunks:
                gds[b] = gather(c + nbuf, b)

    return gather_kernel


@jax.jit
def kernel(word_indexes, W):
    B, L = word_indexes.shape
    V, D = W.shape
    idx_flat = word_indexes.reshape(-1).astype(jnp.int32)
    out = _build(B * L, V, D, 800, 4)(idx_flat, W)
    return out.reshape(B, L, D)


# speedup vs baseline: 1.3452x; 1.3452x over previous
import functools

import jax
import jax.numpy as jnp
from jax import lax
from jax.experimental import pallas as pl
from jax.experimental.pallas import tpu as pltpu
from jax.experimental.pallas import tpu_sc as plsc

NC = 2
NS = 16
NW = NC * NS
LANES = 16


@functools.lru_cache(maxsize=None)
def _build(B: int, L: int, V: int, D: int, nbuf: int):
    assert B % (NW * LANES) == 0
    b_per_w = B // NW
    cb = LANES
    nchunks = b_per_w // cb
    chunk = cb * L
    assert chunk % 8 == 0

    mesh = plsc.VectorSubcoreMesh(
        core_axis_name="c", subcore_axis_name="s",
        num_cores=NC, num_subcores=NS,
    )

    @functools.partial(
        pl.kernel,
        out_type=jax.ShapeDtypeStruct((L, D, B), jnp.float32),
        mesh=mesh,
        scratch_types=[
            pltpu.VMEM((b_per_w * L,), jnp.int32),
            pltpu.VMEM((nbuf, chunk, D), jnp.float32),
            pltpu.VMEM((nbuf, L, D, cb), jnp.float32),
            [pltpu.SemaphoreType.DMA] * nbuf,
            [pltpu.SemaphoreType.DMA] * nbuf,
        ],
        compiler_params=pltpu.CompilerParams(
            use_tc_tiling_on_sc=False, needs_layout_passes=False),
    )
    def gather_kernel(idx_hbm, table_hbm, out_hbm, idx_v, g_v, t_v,
                      gsems, wsems):
        wid = lax.axis_index("s") * NC + lax.axis_index("c")
        base = wid * b_per_w * L
        pltpu.sync_copy(idx_hbm.at[pl.ds(base, b_per_w * L)], idx_v)

        def gather(c, b):
            return pltpu.async_copy(
                table_hbm.at[idx_v.at[pl.ds(c * chunk, chunk)]],
                g_v.at[b], gsems[b])

        def transpose(b):
            def body(l, _):
                rows = lax.iota(jnp.int32, LANES) * L + l
                for d in range(D):
                    cols = jnp.full((LANES,), d, jnp.int32)
                    t_v[b, l, d, :] = plsc.load_gather(
                        g_v.at[b], [rows, cols])
                return _
            lax.fori_loop(0, L, body, None)

        def store(c, b):
            return pltpu.async_copy(
                t_v.at[b],
                out_hbm.at[:, :, pl.ds(wid * b_per_w + c * cb, cb)],
                wsems[b])

        def wait_gather(b):
            pltpu.make_async_copy(
                table_hbm.at[idx_v.at[pl.ds(0, chunk)]],
                g_v.at[b], gsems[b]).wait()

        def wait_store(b):
            pltpu.make_async_copy(
                t_v.at[b], out_hbm.at[:, :, pl.ds(0, cb)], wsems[b]).wait()

        for b in range(nbuf):
            gather(b, b)
        for b in range(nbuf):
            wait_gather(b)
            transpose(b)
            store(b, b)
            gather(b + nbuf, b)

        ngroups = nchunks // nbuf

        def body(g, carry):
            for b in range(nbuf):
                c = g * nbuf + b
                wait_gather(b)
                wait_store(b)
                transpose(b)
                store(c, b)
                gather((c + nbuf) % nchunks, b)
            return carry
        lax.fori_loop(1, ngroups, body, 0)

        for b in range(nbuf):
            wait_gather(b)
            wait_store(b)

    return gather_kernel


@jax.jit
def kernel(word_indexes, W):
    B, L = word_indexes.shape
    V, D = W.shape
    idx_flat = word_indexes.reshape(-1).astype(jnp.int32)
    t = _build(B, L, V, D, 2)(idx_flat, W)
    return jnp.transpose(t, (2, 0, 1))
